# baseline (speedup 1.0000x reference)
import functools

import jax
import jax.numpy as jnp
from jax import lax
from jax.experimental import pallas as pl
from jax.experimental.pallas import tpu as pltpu
from jax.experimental.pallas import tpu_sc as plsc

B = 16384
D = 64
R = 1000
L = 16

_info = plsc.get_sparse_core_info()
NC = _info.num_cores
NS = _info.num_subcores
NW = NC * NS
BPW = B // NW
NG = BPW // L

_mesh = plsc.VectorSubcoreMesh(core_axis_name="c", subcore_axis_name="s")


@functools.partial(
    pl.kernel,
    mesh=_mesh,
    out_type=jax.ShapeDtypeStruct((B,), jnp.float32),
    compiler_params=pltpu.CompilerParams(
        needs_layout_passes=False, use_tc_tiling_on_sc=False),
    scratch_types=[
        pltpu.VMEM((BPW,), jnp.int32),
        pltpu.VMEM((BPW,), jnp.int32),
        pltpu.VMEM((BPW,), jnp.int32),
        pltpu.VMEM((D, R), jnp.float32),
        pltpu.VMEM((L, D, L), jnp.float32),
        pltpu.VMEM((L, D, L), jnp.float32),
        pltpu.VMEM((BPW,), jnp.float32),
        pltpu.SemaphoreType.DMA,
        pltpu.SemaphoreType.DMA,
    ],
)
def _transe_sc(heads_hbm, rels_hbm, tails_hbm, ent_hbm, rel_hbm, out_hbm,
               ridx, hidx, tidx, rtab, hw, tw, outv, sem, sem_r):
    wid = lax.axis_index("s") * NC + lax.axis_index("c")
    base = wid * BPW

    cr = pltpu.async_copy(rel_hbm, rtab, sem_r)
    pltpu.sync_copy(rels_hbm.at[pl.ds(base, BPW)], ridx)
    pltpu.sync_copy(heads_hbm.at[pl.ds(base, BPW)], hidx)
    pltpu.sync_copy(tails_hbm.at[pl.ds(base, BPW)], tidx)
    cr.wait()

    lanes = lax.iota(jnp.int32, L)

    def group_body(g, carry):
        hvec = hidx[pl.ds(g * L, L)]
        tvec = tidx[pl.ds(g * L, L)]
        rvec = ridx[pl.ds(g * L, L)]
        hbase = (hvec // L) * L
        tbase = (tvec // L) * L
        hoff = hvec - hbase
        toff = tvec - tbase
        for k in range(L):
            pltpu.async_copy(
                ent_hbm.at[:, pl.ds(pl.multiple_of(hbase[k], L), L)],
                hw.at[k], sem)
            pltpu.async_copy(
                ent_hbm.at[:, pl.ds(pl.multiple_of(tbase[k], L), L)],
                tw.at[k], sem)
        for k in range(L):
            pltpu.make_async_copy(ent_hbm.at[:, pl.ds(0, L)],
                                  hw.at[k], sem).wait()
            pltpu.make_async_copy(ent_hbm.at[:, pl.ds(0, L)],
                                  tw.at[k], sem).wait()

        acc = jnp.zeros((L,), jnp.float32)
        for d in range(D):
            dvec = jnp.full((L,), d, jnp.int32)
            h = plsc.load_gather(hw, [lanes, dvec, hoff])
            t = plsc.load_gather(tw, [lanes, dvec, toff])
            r = plsc.load_gather(rtab, [dvec, rvec])
            acc = acc + jnp.abs(h + r - t)
        outv[pl.ds(g * L, L)] = acc
        return carry

    lax.fori_loop(0, NG, group_body, 0)

    pltpu.sync_copy(outv, out_hbm.at[pl.ds(base, BPW)])


def kernel(heads, relations, tails, entity_table, relation_table):
    return _transe_sc(heads, relations, tails,
                      entity_table.T, relation_table.T)


# speedup vs baseline: 7.7374x; 7.7374x over previous
import functools

import jax
import jax.numpy as jnp
from jax import lax
from jax.experimental import pallas as pl
from jax.experimental.pallas import tpu as pltpu
from jax.experimental.pallas import tpu_sc as plsc

B = 16384
D = 64
L = 16
P = 2 * D

_info = plsc.get_sparse_core_info()
NC = _info.num_cores
NS = _info.num_subcores
NW = NC * NS
BPW = B // NW
CH = 128
NCH = BPW // CH
NG = CH // L

_mesh = plsc.VectorSubcoreMesh(core_axis_name="c", subcore_axis_name="s")


@functools.partial(
    pl.kernel,
    mesh=_mesh,
    out_type=jax.ShapeDtypeStruct((B,), jnp.float32),
    compiler_params=pltpu.CompilerParams(
        needs_layout_passes=False, use_tc_tiling_on_sc=True),
    scratch_types=[
        pltpu.VMEM((BPW,), jnp.int32),
        pltpu.VMEM((BPW,), jnp.int32),
        pltpu.VMEM((BPW,), jnp.int32),
        pltpu.VMEM((NCH, CH), jnp.int32),
        pltpu.VMEM((NCH, CH), jnp.int32),
        pltpu.VMEM((NCH, CH), jnp.int32),
        pltpu.VMEM((CH, P), jnp.float32),
        pltpu.VMEM((CH, P), jnp.float32),
        pltpu.VMEM((CH, P), jnp.float32),
        pltpu.VMEM((CH, P), jnp.float32),
        pltpu.VMEM((CH, P), jnp.float32),
        pltpu.VMEM((CH, P), jnp.float32),
        pltpu.VMEM((BPW,), jnp.float32),
        pltpu.SemaphoreType.DMA,
        pltpu.SemaphoreType.DMA,
    ],
)
def _transe_sc(heads_hbm, rels_hbm, tails_hbm, ent_hbm, rel_hbm, out_hbm,
               hidx, ridx, tidx, hpix, rpix, tpix,
               hb0, hb1, rb0, rb1, tb0, tb1, outv, sem0, sem1):
    wid = lax.axis_index("s") * NC + lax.axis_index("c")
    base = wid * BPW

    pltpu.sync_copy(heads_hbm.at[pl.ds(base, BPW)], hidx)
    pltpu.sync_copy(rels_hbm.at[pl.ds(base, BPW)], ridx)
    pltpu.sync_copy(tails_hbm.at[pl.ds(base, BPW)], tidx)

    def fill_pix(v, carry):
        sl = pl.ds(v * L, L)
        c = v // NG
        j = (v % NG) * L
        hpix[c, pl.ds(j, L)] = hidx[sl] // 2
        rpix[c, pl.ds(j, L)] = ridx[sl] // 2
        tpix[c, pl.ds(j, L)] = tidx[sl] // 2
        return carry

    for v in range(BPW // L):
        fill_pix(v, 0)

    hbufs = (hb0, hb1)
    rbufs = (rb0, rb1)
    tbufs = (tb0, tb1)
    sems = (sem0, sem1)

    def issue_chunk(c):
        p = c % 2
        return (
            pltpu.async_copy(ent_hbm.at[hpix.at[c]], hbufs[p], sems[p]),
            pltpu.async_copy(rel_hbm.at[rpix.at[c]], rbufs[p], sems[p]),
            pltpu.async_copy(ent_hbm.at[tpix.at[c]], tbufs[p], sems[p]),
        )

    pending = issue_chunk(0)
    lanes = lax.iota(jnp.int32, L)

    for c in range(NCH):
        for cp in pending:
            cp.wait()
        pending = issue_chunk(c + 1) if c + 1 < NCH else ()
        p = c % 2
        hbuf, rbuf, tbuf = hbufs[p], rbufs[p], tbufs[p]

        def group_body(g, carry, c=c, hbuf=hbuf, rbuf=rbuf, tbuf=tbuf):
            sl = pl.ds(c * CH + g * L, L)
            hpar = (hidx[sl] % 2) * D
            rpar = (ridx[sl] % 2) * D
            tpar = (tidx[sl] % 2) * D
            rows = g * L + lanes
            acc = jnp.zeros((L,), jnp.float32)
            for d in range(D):
                h = plsc.load_gather(hbuf, [rows, hpar + d])
                r = plsc.load_gather(rbuf, [rows, rpar + d])
                t = plsc.load_gather(tbuf, [rows, tpar + d])
                acc = acc + jnp.abs(h + r - t)
            outv[pl.ds(c * CH + g * L, L)] = acc
            return carry

        lax.fori_loop(0, NG, group_body, 0)

    pltpu.sync_copy(outv, out_hbm.at[pl.ds(base, BPW)])


def kernel(heads, relations, tails, entity_table, relation_table):
    ent_pairs = entity_table.reshape(entity_table.shape[0] // 2, P)
    rel_pairs = relation_table.reshape(relation_table.shape[0] // 2, P)
    return _transe_sc(heads, relations, tails, ent_pairs, rel_pairs)
